# baseline (speedup 1.0000x reference)
import functools

import jax
import jax.numpy as jnp
from jax import lax
from jax.experimental import pallas as pl
from jax.experimental.pallas import tpu as pltpu
from jax.experimental.pallas import tpu_sc as plsc

_VOCAB = 50257
_N_POS = 2048
_D = 768
_B = 4
_S = 2048
_NW = 32
_SPW = _S // _NW
_LANES = 16
_CHUNK = 16
_NCHUNK = _B * _SPW // _CHUNK
_NSLOT = 4


def _emb_body(ids_hbm, wte_hbm, wpe_hbm, out_hbm,
              idx_v, wpe_v, rows0, rows1, rows2, rows3,
              g0, g1, g2, g3, s0, s1, s2, s3):
    rows = [rows0, rows1, rows2, rows3]
    gsem = [g0, g1, g2, g3]
    ssem = [s0, s1, s2, s3]

    cid = lax.axis_index("c")
    sid = lax.axis_index("s")
    wid = sid * 2 + cid
    s_base = wid * _SPW

    for b in range(_B):
        pltpu.sync_copy(ids_hbm.at[b, pl.ds(s_base, _SPW)], idx_v.at[b])

    def start_gather(c):
        b, h = divmod(c, _SPW // _CHUNK)
        j = c % _NSLOT
        return pltpu.async_copy(
            wte_hbm.at[idx_v.at[b, pl.ds(h * _CHUNK, _CHUNK)]],
            rows[j], gsem[j])

    gathers = {}
    stores = {}
    gathers[0] = start_gather(0)
    gathers[1] = start_gather(1)

    pltpu.sync_copy(wpe_hbm.at[pl.ds(s_base, _SPW)], wpe_v)

    for c in range(_NCHUNK):
        b, h = divmod(c, _SPW // _CHUNK)
        j = c % _NSLOT
        if c + 2 < _NCHUNK:
            if c - 2 >= 0:
                stores[c - 2].wait()
            gathers[c + 2] = start_gather(c + 2)
        gathers[c].wait()

        def row_add(r, carry):
            for col in range(_D // _LANES):
                sl = pl.ds(col * _LANES, _LANES)
                plsc.addupdate(rows[j].at[r, sl], wpe_v[h * _CHUNK + r, sl])
            return carry

        lax.fori_loop(0, _CHUNK, row_add, 0)

        stores[c] = pltpu.async_copy(
            rows[j], out_hbm.at[b, pl.ds(s_base + h * _CHUNK, _CHUNK)],
            ssem[j])

    for c in range(max(0, _NCHUNK - 4), _NCHUNK):
        stores[c].wait()


_emb = functools.partial(
    pl.kernel,
    out_type=jax.ShapeDtypeStruct((_B, _S, _D), jnp.float32),
    mesh=plsc.VectorSubcoreMesh(core_axis_name="c", subcore_axis_name="s"),
    scratch_types=[
        pltpu.VMEM((_B, _SPW), jnp.int32),
        pltpu.VMEM((_SPW, _D), jnp.float32),
        pltpu.VMEM((_CHUNK, _D), jnp.float32),
        pltpu.VMEM((_CHUNK, _D), jnp.float32),
        pltpu.VMEM((_CHUNK, _D), jnp.float32),
        pltpu.VMEM((_CHUNK, _D), jnp.float32),
        pltpu.SemaphoreType.DMA,
        pltpu.SemaphoreType.DMA,
        pltpu.SemaphoreType.DMA,
        pltpu.SemaphoreType.DMA,
        pltpu.SemaphoreType.DMA,
        pltpu.SemaphoreType.DMA,
        pltpu.SemaphoreType.DMA,
        pltpu.SemaphoreType.DMA,
    ],
)(_emb_body)


def kernel(input_ids, wte, wpe):
    ids = jnp.asarray(input_ids, jnp.int32)
    return _emb(ids, wte, wpe)
